# baseline (speedup 1.0000x reference)
import functools

import jax
import jax.numpy as jnp
from jax import lax
from jax.experimental import pallas as pl
from jax.experimental.pallas import tpu as pltpu
from jax.experimental.pallas import tpu_sc as plsc

B = 2
N_IN = 327680
N_OUT = 81920
D = 64

NW = 32
WPB = NW // B
ROWS_PER_W = N_OUT // WPB
CG = 64
NCHUNK = ROWS_PER_W // CG
NBUF = 2
NPAIR = NCHUNK // NBUF
LPR = D // 16


def _make_kernel():
    mesh = plsc.VectorSubcoreMesh(core_axis_name="c", subcore_axis_name="s")

    @functools.partial(
        pl.kernel,
        mesh=mesh,
        out_type=jax.ShapeDtypeStruct((B, N_OUT, D), jnp.float32),
        scratch_types=(
            [pltpu.VMEM((4 * CG, D), jnp.float32) for _ in range(NBUF)]
            + [pltpu.VMEM((CG, D), jnp.float32) for _ in range(NBUF)]
            + [pltpu.SemaphoreType.DMA for _ in range(2 * NBUF)]
        ),
    )
    def k(x_hbm, out_hbm, in0, in1, ob0, ob1, isem0, isem1, osem0, osem1):
        ibufs = (in0, in1)
        obufs = (ob0, ob1)
        isems = (isem0, isem1)
        osems = (osem0, osem1)
        wid = lax.axis_index("s") * 2 + lax.axis_index("c")
        b = wid // WPB
        base = (wid % WPB) * ROWS_PER_W

        def in_copy(ci, slot):
            off = base + ci * CG
            return pltpu.make_async_copy(
                x_hbm.at[b, pl.ds(4 * off, 4 * CG)], ibufs[slot], isems[slot]
            )

        def out_copy(ci, slot):
            off = base + ci * CG
            return pltpu.make_async_copy(
                obufs[slot], out_hbm.at[b, pl.ds(off, CG)], osems[slot]
            )

        def select(slot):
            src = ibufs[slot]
            dst = obufs[slot]

            def body(i, _):
                for u in range(2):
                    r = 2 * i + u
                    for c in range(LPR):
                        dst[r, pl.ds(16 * c, 16)] = src[4 * r + 3, pl.ds(16 * c, 16)]
                return 0

            lax.fori_loop(0, CG // 2, body, 0)

        def step(ci, slot, start_next):
            in_copy(ci, slot).wait()
            select(slot)
            out_copy(ci, slot).start()
            out_copy(ci, slot).wait()
            if start_next:
                in_copy(ci + NBUF, slot).start()

        for s in range(NBUF):
            in_copy(s, s).start()

        def pair(g, _):
            for s in range(NBUF):
                step(g * NBUF + s, s, True)
            return 0

        lax.fori_loop(0, NPAIR - 1, pair, 0)

        for s in range(NBUF):
            step((NPAIR - 1) * NBUF + s, s, False)

    return k


_sc_copy = _make_kernel()


@jax.jit
def kernel(x):
    return _sc_copy(x)
